# baseline (speedup 1.0000x reference)
import functools

import jax
import jax.numpy as jnp
from jax import lax
from jax.experimental import pallas as pl
from jax.experimental.pallas import tpu as pltpu
from jax.experimental.pallas import tpu_sc as plsc

_THRESH = [2.0919, 1.5026, 1.6009, 2.1762, 2.326, 2.1743, 2.0768, 1.9951,
           2.0089, 1.9652, 2.1529, 2.5862, 2.6576, 2.5778, 2.7211, 2.59]

_B, _C, _H, _W, _K = 8, 17, 256, 256, 36
_HW = _H * _W
_KP = 48
_ROWS = _B * _C * _H
_N = _ROWS * _W

_SC_ROWS = 16384
_TC_ROWS = _ROWS - _SC_ROWS
_TSTEPS = 4
_BR = _TC_ROWS // _TSTEPS
_NSUB = 32
_SUB_ROWS = _SC_ROWS // _NSUB
_PIECES = 8
_PR = _SUB_ROWS // _PIECES
_SC_ROW_BASE = _TC_ROWS

_LN50 = 3.9120230054281460
_PI = 3.14159265358979323846


def _sc_gather(vp4, ind_pad):
    mesh = plsc.VectorSubcoreMesh(core_axis_name="c", subcore_axis_name="s")

    @functools.partial(
        pl.kernel,
        mesh=mesh,
        compiler_params=pltpu.CompilerParams(use_tc_tiling_on_sc=False),
        out_type=jax.ShapeDtypeStruct((16, _KP, 16), jnp.float32),
        scratch_types=[
            pltpu.VMEM((_KP,), jnp.int32),
            pltpu.VMEM((_KP,), jnp.int32),
            pltpu.VMEM((_KP, 16), jnp.float32),
            pltpu.SemaphoreType.DMA,
        ],
    )
    def k(vp_hbm, ind_hbm, rows_out, ind_v, ridx_v, rows_v, sem):
        wid = lax.axis_index("s") * 2 + lax.axis_index("c")

        @pl.when(wid < 16)
        def _():
            ch = wid // 8
            b = wid - ch * 8
            pltpu.sync_copy(ind_hbm.at[b], ind_v)
            base = (b * 2 + ch) * (_HW // 16)
            for j in range(_KP // 16):
                sl = pl.ds(j * 16, 16)
                ridx_v[sl] = lax.shift_right_logical(ind_v[sl], 4) + base
            pltpu.async_copy(vp_hbm.at[ridx_v], rows_v, sem).wait()
            pltpu.sync_copy(rows_v, rows_out.at[wid])

    return k(vp4, ind_pad)


def _sc_mse(hp2, hg2):
    mesh = plsc.VectorSubcoreMesh(core_axis_name="c", subcore_axis_name="s")

    @functools.partial(
        pl.kernel,
        mesh=mesh,
        out_type=jax.ShapeDtypeStruct((_NSUB, 16), jnp.float32),
        scratch_types=[
            pltpu.VMEM((_PR, _W), jnp.float32),
            pltpu.VMEM((_PR, _W), jnp.float32),
            pltpu.VMEM((16,), jnp.float32),
        ],
    )
    def k(hp_hbm, hg_hbm, part_out, hp_v, hg_v, acc_v):
        wid = lax.axis_index("s") * 2 + lax.axis_index("c")
        row0 = _SC_ROW_BASE + wid * _SUB_ROWS
        accs = (jnp.zeros((16,), jnp.float32),) * 4
        for p in range(_PIECES):
            r0 = row0 + p * _PR
            pltpu.sync_copy(hp_hbm.at[pl.ds(r0, _PR)], hp_v)
            pltpu.sync_copy(hg_hbm.at[pl.ds(r0, _PR)], hg_v)

            def body(r, a):
                res = list(a)
                for c in range(_W // 16):
                    sl = pl.ds(c * 16, 16)
                    pv = hp_v[r, sl]
                    gv = hg_v[r, sl]
                    d = pv - gv
                    res[c % 4] = res[c % 4] + d * d * jnp.exp(gv * _LN50)
                return tuple(res)

            accs = lax.fori_loop(0, _PR, body, accs)
        acc_v[...] = (accs[0] + accs[1]) + (accs[2] + accs[3])
        pltpu.sync_copy(acc_v, part_out.at[wid])

    return k(hp2, hg2)


def _tc_reduce_body(hp_ref, hg_ref, out_ref):
    i = pl.program_id(0)

    @pl.when(i == 0)
    def _():
        out_ref[0] = 0.0

    hp = hp_ref[...]
    hg = hg_ref[...]
    d = hp - hg
    w = jnp.exp(hg * _LN50)
    out_ref[0] += jnp.sum(d * d * w)


def _combine_body(thr_ref, rows_ref, ind_ref, vgx_ref, vgy_ref, mask_ref,
                  ppx_ref, ppy_ref, scp_ref, tcp_ref, out_ref):
    rem = lax.bitwise_and(ind_ref[...], 15)
    iota3 = lax.broadcasted_iota(jnp.int32, (16, _KP, 16), 2)
    vals = jnp.sum(jnp.where(iota3 == rem[..., None], rows_ref[...], 0.0),
                   axis=2)
    px = vals[0:_B, 0:_K]
    py = vals[_B:2 * _B, 0:_K]
    num = jnp.sum(jnp.abs(px - vgx_ref[...]) + jnp.abs(py - vgy_ref[...]))
    den = jnp.sum(mask_ref[...]) + 0.0001
    vec_loss = num / den
    tu = jnp.arctan2(py[:, :18], px[:, :18])
    tl = jnp.arctan2(py[:, 18:], px[:, 18:])
    thetas = 0.5 * (tu + tl)
    ppx = ppx_ref[...]
    ppy = ppy_ref[...]
    cvx = ppx[:, 1:] - ppx[:, :-1]
    cvy = ppy[:, 1:] - ppy[:, :-1]
    betas = jnp.arctan2(cvy, cvx)
    nxt = jnp.concatenate([betas[:, 1:], betas[:, 15:16]], axis=1)
    beta_avg = 0.5 * (betas + nxt) + 0.5 * _PI
    dlt = beta_avg - thetas[:, 1:17]
    dlt = jnp.where(dlt > _PI, dlt - 2.0 * _PI, dlt)
    dlt = jnp.where(dlt <= -_PI, dlt + 2.0 * _PI, dlt)
    loss = jnp.abs(dlt)
    loss = loss * (loss > thr_ref[...]).astype(jnp.float32)
    constraint = jnp.sum(loss) * (1.0 / (16.0 * _B))
    hm_loss = (jnp.sum(scp_ref[...]) + tcp_ref[0]) * (1.0 / _N)
    out_ref[0] = hm_loss + vec_loss + 0.5 * constraint
    out_ref[1] = hm_loss
    out_ref[2] = vec_loss
    out_ref[3] = constraint


def kernel(hm_pred, hm_gt, vec_pred, ind, vec_gt, reg_mask, peak_points):
    vp4 = vec_pred.reshape(_HW, 16)
    ind_pad = jnp.pad(ind, ((0, 0), (0, _KP - _K)))
    rows = _sc_gather(vp4, ind_pad)
    hp2 = hm_pred.reshape(_ROWS, _W)
    hg2 = hm_gt.reshape(_ROWS, _W)

    tcpart = pl.pallas_call(
        _tc_reduce_body,
        grid=(_TSTEPS,),
        in_specs=[
            pl.BlockSpec((_BR, _W), lambda i: (i, 0)),
            pl.BlockSpec((_BR, _W), lambda i: (i, 0)),
        ],
        out_specs=pl.BlockSpec(memory_space=pltpu.SMEM, block_shape=(1,),
                               index_map=lambda i: (0,)),
        out_shape=jax.ShapeDtypeStruct((1,), jnp.float32),
    )(hp2, hg2)
    scpart = _sc_mse(hp2, hg2)

    ind16 = jnp.concatenate([ind_pad, ind_pad], axis=0)
    vgx = vec_gt[:, :, 0]
    vgy = vec_gt[:, :, 1]
    ppx = peak_points[:, :, 0]
    ppy = peak_points[:, :, 1]
    thr = jnp.asarray(_THRESH, dtype=jnp.float32).reshape(1, 16) * (_PI / 180.0)

    out = pl.pallas_call(
        _combine_body,
        in_specs=[
            pl.BlockSpec((1, 16), lambda: (0, 0)),
            pl.BlockSpec((16, _KP, 16), lambda: (0, 0, 0)),
            pl.BlockSpec((16, _KP), lambda: (0, 0)),
            pl.BlockSpec((_B, _K), lambda: (0, 0)),
            pl.BlockSpec((_B, _K), lambda: (0, 0)),
            pl.BlockSpec((_B, _K), lambda: (0, 0)),
            pl.BlockSpec((_B, _C), lambda: (0, 0)),
            pl.BlockSpec((_B, _C), lambda: (0, 0)),
            pl.BlockSpec((_NSUB, 16), lambda: (0, 0)),
            pl.BlockSpec(memory_space=pltpu.SMEM, block_shape=(1,),
                         index_map=lambda: (0,)),
        ],
        out_specs=pl.BlockSpec(memory_space=pltpu.SMEM, block_shape=(4,),
                               index_map=lambda: (0,)),
        out_shape=jax.ShapeDtypeStruct((4,), jnp.float32),
    )(thr, rows, ind16, vgx, vgy, reg_mask, ppx, ppy, scpart, tcpart)

    return (out[0], out[1], out[2], out[3])


# speedup vs baseline: 1.1232x; 1.1232x over previous
import functools

import jax
import jax.numpy as jnp
from jax import lax
from jax.experimental import pallas as pl
from jax.experimental.pallas import tpu as pltpu
from jax.experimental.pallas import tpu_sc as plsc

_THRESH = [2.0919, 1.5026, 1.6009, 2.1762, 2.326, 2.1743, 2.0768, 1.9951,
           2.0089, 1.9652, 2.1529, 2.5862, 2.6576, 2.5778, 2.7211, 2.59]

_B, _C, _H, _W, _K = 8, 17, 256, 256, 36
_HW = _H * _W
_KP = 48
_ROWS = _B * _C * _H
_N = _ROWS * _W

_SC_ROWS = 8192
_TC_ROWS = _ROWS - _SC_ROWS
_TSTEPS = 4
_BR = _TC_ROWS // _TSTEPS
_NSUB = 32
_SUB_ROWS = _SC_ROWS // _NSUB
_PIECES = 8
_PR = _SUB_ROWS // _PIECES
_SC_ROW_BASE = _TC_ROWS

_LN50 = 3.9120230054281460
_PI = 3.14159265358979323846


def _sc_gather(vp128, ind_pad):
    mesh = plsc.VectorSubcoreMesh(core_axis_name="c", subcore_axis_name="s")

    @functools.partial(
        pl.kernel,
        mesh=mesh,
        compiler_params=pltpu.CompilerParams(use_tc_tiling_on_sc=False),
        out_type=jax.ShapeDtypeStruct((16, _KP, 128), jnp.float32),
        scratch_types=[
            pltpu.VMEM((_KP,), jnp.int32),
            pltpu.VMEM((_KP,), jnp.int32),
            pltpu.VMEM((_KP, 128), jnp.float32),
            pltpu.SemaphoreType.DMA,
        ],
    )
    def k(vp_hbm, ind_hbm, rows_out, ind_v, ridx_v, rows_v, sem):
        wid = lax.axis_index("s") * 2 + lax.axis_index("c")

        @pl.when(wid < 16)
        def _():
            ch = wid // 8
            b = wid - ch * 8
            pltpu.sync_copy(ind_hbm.at[b], ind_v)
            base = (b * 2 + ch) * (_HW // 128)
            for j in range(_KP // 16):
                sl = pl.ds(j * 16, 16)
                ridx_v[sl] = lax.shift_right_logical(ind_v[sl], 7) + base
            pltpu.async_copy(vp_hbm.at[ridx_v], rows_v, sem).wait()
            pltpu.sync_copy(rows_v, rows_out.at[wid])

    return k(vp128, ind_pad)


def _sc_mse(hp2, hg2):
    mesh = plsc.VectorSubcoreMesh(core_axis_name="c", subcore_axis_name="s")

    @functools.partial(
        pl.kernel,
        mesh=mesh,
        out_type=jax.ShapeDtypeStruct((_NSUB, 16), jnp.float32),
        scratch_types=[
            pltpu.VMEM((_PR, _W), jnp.float32),
            pltpu.VMEM((_PR, _W), jnp.float32),
            pltpu.VMEM((16,), jnp.float32),
        ],
    )
    def k(hp_hbm, hg_hbm, part_out, hp_v, hg_v, acc_v):
        wid = lax.axis_index("s") * 2 + lax.axis_index("c")
        row0 = _SC_ROW_BASE + wid * _SUB_ROWS
        accs = (jnp.zeros((16,), jnp.float32),) * 4
        for p in range(_PIECES):
            r0 = row0 + p * _PR
            pltpu.sync_copy(hp_hbm.at[pl.ds(r0, _PR)], hp_v)
            pltpu.sync_copy(hg_hbm.at[pl.ds(r0, _PR)], hg_v)

            def body(r, a):
                res = list(a)
                for c in range(_W // 16):
                    sl = pl.ds(c * 16, 16)
                    pv = hp_v[r, sl]
                    gv = hg_v[r, sl]
                    d = pv - gv
                    res[c % 4] = res[c % 4] + d * d * jnp.exp(gv * _LN50)
                return tuple(res)

            accs = lax.fori_loop(0, _PR, body, accs)
        acc_v[...] = (accs[0] + accs[1]) + (accs[2] + accs[3])
        pltpu.sync_copy(acc_v, part_out.at[wid])

    return k(hp2, hg2)


def _tc_reduce_body(hp_ref, hg_ref, out_ref):
    i = pl.program_id(0)

    @pl.when(i == 0)
    def _():
        out_ref[0] = 0.0

    hp = hp_ref[...]
    hg = hg_ref[...]
    d = hp - hg
    w = jnp.exp(hg * _LN50)
    out_ref[0] += jnp.sum(d * d * w)


def _combine_body(thr_ref, rows_ref, ind_ref, vgx_ref, vgy_ref, mask_ref,
                  ppx_ref, ppy_ref, scp_ref, tcp_ref, out_ref):
    rem = lax.bitwise_and(ind_ref[...], 127)
    iota3 = lax.broadcasted_iota(jnp.int32, (16, _KP, 128), 2)
    vals = jnp.sum(jnp.where(iota3 == rem[..., None], rows_ref[...], 0.0),
                   axis=2)
    px = vals[0:_B, 0:_K]
    py = vals[_B:2 * _B, 0:_K]
    num = jnp.sum(jnp.abs(px - vgx_ref[...]) + jnp.abs(py - vgy_ref[...]))
    den = jnp.sum(mask_ref[...]) + 0.0001
    vec_loss = num / den
    tu = jnp.arctan2(py[:, :18], px[:, :18])
    tl = jnp.arctan2(py[:, 18:], px[:, 18:])
    thetas = 0.5 * (tu + tl)
    ppx = ppx_ref[...]
    ppy = ppy_ref[...]
    cvx = ppx[:, 1:] - ppx[:, :-1]
    cvy = ppy[:, 1:] - ppy[:, :-1]
    betas = jnp.arctan2(cvy, cvx)
    nxt = jnp.concatenate([betas[:, 1:], betas[:, 15:16]], axis=1)
    beta_avg = 0.5 * (betas + nxt) + 0.5 * _PI
    dlt = beta_avg - thetas[:, 1:17]
    dlt = jnp.where(dlt > _PI, dlt - 2.0 * _PI, dlt)
    dlt = jnp.where(dlt <= -_PI, dlt + 2.0 * _PI, dlt)
    loss = jnp.abs(dlt)
    loss = loss * (loss > thr_ref[...]).astype(jnp.float32)
    constraint = jnp.sum(loss) * (1.0 / (16.0 * _B))
    hm_loss = (jnp.sum(scp_ref[...]) + tcp_ref[0]) * (1.0 / _N)
    out_ref[0] = hm_loss + vec_loss + 0.5 * constraint
    out_ref[1] = hm_loss
    out_ref[2] = vec_loss
    out_ref[3] = constraint


def kernel(hm_pred, hm_gt, vec_pred, ind, vec_gt, reg_mask, peak_points):
    vp128 = vec_pred.reshape(_HW // 8, 128)
    ind_pad = jnp.pad(ind, ((0, 0), (0, _KP - _K)))
    rows = _sc_gather(vp128, ind_pad)
    hp2 = hm_pred.reshape(_ROWS, _W)
    hg2 = hm_gt.reshape(_ROWS, _W)

    tcpart = pl.pallas_call(
        _tc_reduce_body,
        grid=(_TSTEPS,),
        in_specs=[
            pl.BlockSpec((_BR, _W), lambda i: (i, 0)),
            pl.BlockSpec((_BR, _W), lambda i: (i, 0)),
        ],
        out_specs=pl.BlockSpec(memory_space=pltpu.SMEM, block_shape=(1,),
                               index_map=lambda i: (0,)),
        out_shape=jax.ShapeDtypeStruct((1,), jnp.float32),
    )(hp2, hg2)
    scpart = _sc_mse(hp2, hg2)

    ind16 = jnp.concatenate([ind_pad, ind_pad], axis=0)
    vgx = vec_gt[:, :, 0]
    vgy = vec_gt[:, :, 1]
    ppx = peak_points[:, :, 0]
    ppy = peak_points[:, :, 1]
    thr = jnp.asarray(_THRESH, dtype=jnp.float32).reshape(1, 16) * (_PI / 180.0)

    out = pl.pallas_call(
        _combine_body,
        in_specs=[
            pl.BlockSpec((1, 16), lambda: (0, 0)),
            pl.BlockSpec((16, _KP, 128), lambda: (0, 0, 0)),
            pl.BlockSpec((16, _KP), lambda: (0, 0)),
            pl.BlockSpec((_B, _K), lambda: (0, 0)),
            pl.BlockSpec((_B, _K), lambda: (0, 0)),
            pl.BlockSpec((_B, _K), lambda: (0, 0)),
            pl.BlockSpec((_B, _C), lambda: (0, 0)),
            pl.BlockSpec((_B, _C), lambda: (0, 0)),
            pl.BlockSpec((_NSUB, 16), lambda: (0, 0)),
            pl.BlockSpec(memory_space=pltpu.SMEM, block_shape=(1,),
                         index_map=lambda: (0,)),
        ],
        out_specs=pl.BlockSpec(memory_space=pltpu.SMEM, block_shape=(4,),
                               index_map=lambda: (0,)),
        out_shape=jax.ShapeDtypeStruct((4,), jnp.float32),
    )(thr, rows, ind16, vgx, vgy, reg_mask, ppx, ppy, scpart, tcpart)

    return (out[0], out[1], out[2], out[3])


# speedup vs baseline: 1.5474x; 1.3776x over previous
import functools

import jax
import jax.numpy as jnp
from jax import lax
from jax.experimental import pallas as pl
from jax.experimental.pallas import tpu as pltpu
from jax.experimental.pallas import tpu_sc as plsc

_THRESH = [2.0919, 1.5026, 1.6009, 2.1762, 2.326, 2.1743, 2.0768, 1.9951,
           2.0089, 1.9652, 2.1529, 2.5862, 2.6576, 2.5778, 2.7211, 2.59]

_B, _C, _H, _W, _K = 8, 17, 256, 256, 36
_HW = _H * _W
_KP = 48
_ROWS = _B * _C * _H
_N = _ROWS * _W

_TSTEPS = 8
_BR = _ROWS // _TSTEPS

_LN50 = 3.9120230054281460
_PI = 3.14159265358979323846


def _sc_gather(vp2, ind_pad):
    mesh = plsc.VectorSubcoreMesh(core_axis_name="c", subcore_axis_name="s")

    @functools.partial(
        pl.kernel,
        mesh=mesh,
        out_type=jax.ShapeDtypeStruct((16, _KP, 256), jnp.float32),
        scratch_types=[
            pltpu.VMEM((_KP,), jnp.int32),
            pltpu.VMEM((_KP,), jnp.int32),
            pltpu.VMEM((_KP, 256), jnp.float32),
            pltpu.SemaphoreType.DMA,
        ],
    )
    def k(vp_hbm, ind_hbm, rows_out, ind_v, ridx_v, rows_v, sem):
        wid = lax.axis_index("s") * 2 + lax.axis_index("c")

        @pl.when(wid < 16)
        def _():
            ch = wid // 8
            b = wid - ch * 8
            pltpu.sync_copy(ind_hbm.at[b], ind_v)
            base = (b * 2 + ch) * _H
            for j in range(_KP // 16):
                sl = pl.ds(j * 16, 16)
                ridx_v[sl] = lax.shift_right_logical(ind_v[sl], 8) + base
            pltpu.async_copy(vp_hbm.at[ridx_v], rows_v, sem).wait()
            pltpu.sync_copy(rows_v, rows_out.at[wid])

    return k(vp2, ind_pad)


def _tc_reduce_body(hp_ref, hg_ref, out_ref):
    i = pl.program_id(0)

    @pl.when(i == 0)
    def _():
        out_ref[0] = 0.0

    hp = hp_ref[...]
    hg = hg_ref[...]
    d = hp - hg
    w = jnp.exp(hg * _LN50)
    out_ref[0] += jnp.sum(d * d * w)


def _combine_body(thr_ref, rows_ref, ind_ref, vgx_ref, vgy_ref, mask_ref,
                  ppx_ref, ppy_ref, tcp_ref, out_ref):
    rem = lax.bitwise_and(ind_ref[...], 255)
    iota3 = lax.broadcasted_iota(jnp.int32, (16, _KP, 256), 2)
    vals = jnp.sum(jnp.where(iota3 == rem[..., None], rows_ref[...], 0.0),
                   axis=2)
    px = vals[0:_B, 0:_K]
    py = vals[_B:2 * _B, 0:_K]
    num = jnp.sum(jnp.abs(px - vgx_ref[...]) + jnp.abs(py - vgy_ref[...]))
    den = jnp.sum(mask_ref[...]) + 0.0001
    vec_loss = num / den
    tu = jnp.arctan2(py[:, :18], px[:, :18])
    tl = jnp.arctan2(py[:, 18:], px[:, 18:])
    thetas = 0.5 * (tu + tl)
    ppx = ppx_ref[...]
    ppy = ppy_ref[...]
    cvx = ppx[:, 1:] - ppx[:, :-1]
    cvy = ppy[:, 1:] - ppy[:, :-1]
    betas = jnp.arctan2(cvy, cvx)
    nxt = jnp.concatenate([betas[:, 1:], betas[:, 15:16]], axis=1)
    beta_avg = 0.5 * (betas + nxt) + 0.5 * _PI
    dlt = beta_avg - thetas[:, 1:17]
    dlt = jnp.where(dlt > _PI, dlt - 2.0 * _PI, dlt)
    dlt = jnp.where(dlt <= -_PI, dlt + 2.0 * _PI, dlt)
    loss = jnp.abs(dlt)
    loss = loss * (loss > thr_ref[...]).astype(jnp.float32)
    constraint = jnp.sum(loss) * (1.0 / (16.0 * _B))
    hm_loss = tcp_ref[0] * (1.0 / _N)
    out_ref[0] = hm_loss + vec_loss + 0.5 * constraint
    out_ref[1] = hm_loss
    out_ref[2] = vec_loss
    out_ref[3] = constraint


def kernel(hm_pred, hm_gt, vec_pred, ind, vec_gt, reg_mask, peak_points):
    vp2 = vec_pred.reshape(_B * 2 * _H, _W)
    ind_pad = jnp.pad(ind, ((0, 0), (0, _KP - _K)))
    rows = _sc_gather(vp2, ind_pad)
    hp2 = hm_pred.reshape(_ROWS, _W)
    hg2 = hm_gt.reshape(_ROWS, _W)

    tcpart = pl.pallas_call(
        _tc_reduce_body,
        grid=(_TSTEPS,),
        in_specs=[
            pl.BlockSpec((_BR, _W), lambda i: (i, 0)),
            pl.BlockSpec((_BR, _W), lambda i: (i, 0)),
        ],
        out_specs=pl.BlockSpec(memory_space=pltpu.SMEM, block_shape=(1,),
                               index_map=lambda i: (0,)),
        out_shape=jax.ShapeDtypeStruct((1,), jnp.float32),
    )(hp2, hg2)

    ind16 = jnp.concatenate([ind_pad, ind_pad], axis=0)
    vgx = vec_gt[:, :, 0]
    vgy = vec_gt[:, :, 1]
    ppx = peak_points[:, :, 0]
    ppy = peak_points[:, :, 1]
    thr = jnp.asarray(_THRESH, dtype=jnp.float32).reshape(1, 16) * (_PI / 180.0)

    out = pl.pallas_call(
        _combine_body,
        in_specs=[
            pl.BlockSpec((1, 16), lambda: (0, 0)),
            pl.BlockSpec((16, _KP, 256), lambda: (0, 0, 0)),
            pl.BlockSpec((16, _KP), lambda: (0, 0)),
            pl.BlockSpec((_B, _K), lambda: (0, 0)),
            pl.BlockSpec((_B, _K), lambda: (0, 0)),
            pl.BlockSpec((_B, _K), lambda: (0, 0)),
            pl.BlockSpec((_B, _C), lambda: (0, 0)),
            pl.BlockSpec((_B, _C), lambda: (0, 0)),
            pl.BlockSpec(memory_space=pltpu.SMEM, block_shape=(1,),
                         index_map=lambda: (0,)),
        ],
        out_specs=pl.BlockSpec(memory_space=pltpu.SMEM, block_shape=(4,),
                               index_map=lambda: (0,)),
        out_shape=jax.ShapeDtypeStruct((4,), jnp.float32),
    )(thr, rows, ind16, vgx, vgy, reg_mask, ppx, ppy, tcpart)

    return (out[0], out[1], out[2], out[3])
